# baseline (speedup 1.0000x reference)
import functools

import jax
import jax.numpy as jnp
from jax import lax
from jax.experimental import pallas as pl
from jax.experimental.pallas import tpu as pltpu
from jax.experimental.pallas import tpu_sc as plsc

_MAX_LEN = 10000
_B, _L, _D = 4096, 200, 64
_N_TOK = _B * _L

_NC, _NS, _LANES = 2, 16, 16
_NW = _NC * _NS
_COLS = _B // _NW


def _gather_body(pe_hbm, idx_hbm, g_hbm, pe_v, idx_v, g_v):
    wid = lax.axis_index("s") * _NC + lax.axis_index("c")
    base = wid * _COLS
    pltpu.sync_copy(pe_hbm, pe_v)
    pltpu.sync_copy(idx_hbm.at[:, pl.ds(base, _COLS)], idx_v)

    @plsc.parallel_loop(0, _L, 1, unroll=2)
    def _row(r):
        for c in range(_COLS // _LANES):
            off = c * _LANES
            iv = idx_v[r, pl.ds(off, _LANES)]
            g_v[r, pl.ds(off, _LANES)] = plsc.load_gather(pe_v, [iv])

    pltpu.sync_copy(g_v, g_hbm.at[:, pl.ds(base, _COLS)])


_gather_sc = functools.partial(
    pl.kernel,
    out_type=jax.ShapeDtypeStruct((_L, _B), jnp.float32),
    mesh=plsc.VectorSubcoreMesh(core_axis_name="c", subcore_axis_name="s"),
    scratch_types=[
        pltpu.VMEM((_MAX_LEN,), jnp.float32),
        pltpu.VMEM((_L, _COLS), jnp.int32),
        pltpu.VMEM((_L, _COLS), jnp.float32),
    ],
    compiler_params=pltpu.CompilerParams(
        needs_layout_passes=False, use_tc_tiling_on_sc=True
    ),
)(_gather_body)


_LB = 8


def _add_body(x_ref, g_ref, o_ref):
    i = pl.program_id(0)
    off = pl.multiple_of(i * _LB, 8)
    g_blk = g_ref[pl.ds(off, _LB), :]
    o_ref[...] = x_ref[...] + g_blk[:, None, :]


def _add_tc(xt, g2):
    return pl.pallas_call(
        _add_body,
        out_shape=jax.ShapeDtypeStruct((_L, _D, _B), jnp.float32),
        grid=(_L // _LB,),
        in_specs=[
            pl.BlockSpec((_LB, _D, _B), lambda i: (i, 0, 0)),
            pl.BlockSpec((_L, _B), lambda i: (0, 0)),
        ],
        out_specs=pl.BlockSpec((_LB, _D, _B), lambda i: (i, 0, 0)),
        compiler_params=pltpu.CompilerParams(vmem_limit_bytes=50 * 1024 * 1024),
    )(xt, g2)


def kernel(x, idx, pe):
    pe1 = pe.reshape(_MAX_LEN)
    xt = jnp.transpose(x, (1, 2, 0))
    idx_t = jnp.transpose(idx)
    g2 = _gather_sc(pe1, idx_t)
    out_t = _add_tc(xt, g2)
    return jnp.transpose(out_t, (2, 0, 1))


# speedup vs baseline: 1.0054x; 1.0054x over previous
import functools

import jax
import jax.numpy as jnp
from jax import lax
from jax.experimental import pallas as pl
from jax.experimental.pallas import tpu as pltpu
from jax.experimental.pallas import tpu_sc as plsc

_MAX_LEN = 10000
_B, _L, _D = 4096, 200, 64
_N_TOK = _B * _L

_NC, _NS, _LANES = 2, 16, 16
_NW = _NC * _NS
_COLS = _B // _NW


_L_HALF = 96


def _gather_body(pe_hbm, idx_hbm, g_hbm, pe_v, idx_v, g_v, sem_pe, sem_in, sem_out):
    wid = lax.axis_index("s") * _NC + lax.axis_index("c")
    base = wid * _COLS
    cp_pe = pltpu.make_async_copy(pe_hbm, pe_v, sem_pe)
    cp_pe.start()
    cp_in = pltpu.make_async_copy(idx_hbm.at[:, pl.ds(base, _COLS)], idx_v, sem_in)
    cp_in.start()
    cp_pe.wait()
    cp_in.wait()

    def _gather_rows(lo, hi):
        @plsc.parallel_loop(lo, hi, 1, unroll=2)
        def _row(r):
            for c in range(_COLS // _LANES):
                off = c * _LANES
                iv = idx_v[r, pl.ds(off, _LANES)]
                g_v[r, pl.ds(off, _LANES)] = plsc.load_gather(pe_v, [iv])

    _gather_rows(0, _L_HALF)
    cp_out0 = pltpu.make_async_copy(
        g_v.at[pl.ds(0, _L_HALF), :], g_hbm.at[pl.ds(0, _L_HALF), pl.ds(base, _COLS)],
        sem_out,
    )
    cp_out0.start()
    _gather_rows(_L_HALF, _L)
    cp_out0.wait()
    pltpu.sync_copy(
        g_v.at[pl.ds(_L_HALF, _L - _L_HALF), :],
        g_hbm.at[pl.ds(_L_HALF, _L - _L_HALF), pl.ds(base, _COLS)],
    )


_gather_sc = functools.partial(
    pl.kernel,
    out_type=jax.ShapeDtypeStruct((_L, _B), jnp.float32),
    mesh=plsc.VectorSubcoreMesh(core_axis_name="c", subcore_axis_name="s"),
    scratch_types=[
        pltpu.VMEM((_MAX_LEN,), jnp.float32),
        pltpu.VMEM((_L, _COLS), jnp.int32),
        pltpu.VMEM((_L, _COLS), jnp.float32),
        pltpu.SemaphoreType.DMA,
        pltpu.SemaphoreType.DMA,
        pltpu.SemaphoreType.DMA,
    ],
    compiler_params=pltpu.CompilerParams(
        needs_layout_passes=False, use_tc_tiling_on_sc=True
    ),
)(_gather_body)


_LB = 8


def _add_body(x_ref, g_ref, o_ref):
    i = pl.program_id(0)
    off = pl.multiple_of(i * _LB, 8)
    g_blk = g_ref[pl.ds(off, _LB), :]
    o_ref[...] = x_ref[...] + g_blk[:, None, :]


def _add_tc(xt, g2):
    return pl.pallas_call(
        _add_body,
        out_shape=jax.ShapeDtypeStruct((_L, _D, _B), jnp.float32),
        grid=(_L // _LB,),
        in_specs=[
            pl.BlockSpec((_LB, _D, _B), lambda i: (i, 0, 0)),
            pl.BlockSpec((_L, _B), lambda i: (0, 0)),
        ],
        out_specs=pl.BlockSpec((_LB, _D, _B), lambda i: (i, 0, 0)),
        compiler_params=pltpu.CompilerParams(vmem_limit_bytes=50 * 1024 * 1024),
    )(xt, g2)


def kernel(x, idx, pe):
    pe1 = pe.reshape(_MAX_LEN)
    xt = jnp.transpose(x, (1, 2, 0))
    idx_t = jnp.transpose(idx)
    g2 = _gather_sc(pe1, idx_t)
    out_t = _add_tc(xt, g2)
    return jnp.transpose(out_t, (2, 0, 1))
